# baseline (speedup 1.0000x reference)
import functools

import jax
import jax.numpy as jnp
from jax import lax
from jax.experimental import pallas as pl
from jax.experimental.pallas import tpu as pltpu
from jax.experimental.pallas import tpu_sc as plsc

N_NODES = 10000
N_EDGES = 320000
IN_CH = 128
HID = 16
OUT_CH = 128

NC = 2
NS = 16
NW = NC * NS
L = 16

K = 128
CPT = 80
E_PAD = NW * CPT * K
NCHUNKS = E_PAD // K
G_ROWS = N_NODES + 8
PAD_SRC = N_NODES
ACC_ROWS = 10112
PAD_DST = N_NODES + 8
HRNG = ACC_ROWS // 2
ACCH = HRNG * HID // L


def _deg_body(dst2_hbm, deg_hbm, dbuf, hist):
    c = lax.axis_index("c")
    s = lax.axis_index("s")
    wid = c * NS + s

    @pl.loop(0, ACC_ROWS // L)
    def _zero(i):
        hist[pl.ds(i * L, L)] = jnp.zeros((L,), jnp.float32)

    pltpu.sync_copy(dst2_hbm.at[pl.ds(wid * CPT, CPT)], dbuf)

    @pl.loop(0, CPT)
    def _chunk(ci):
        for gi in range(K // L):
            idx = dbuf[ci, pl.ds(gi * L, L)]
            plsc.addupdate_scatter(hist, [idx], jnp.full((L,), 1.0, jnp.float32))

    pltpu.sync_copy(hist, deg_hbm.at[wid])


def _agg_body(g_hbm, src2_hbm, dst2_hbm, out_hbm, sbuf, dbuf, rows0, rows1,
              accf, gsem0, gsem1):
    c = lax.axis_index("c")
    s = lax.axis_index("s")
    wid = c * NS + s

    pltpu.sync_copy(src2_hbm.at[pl.ds(wid * CPT, CPT)], sbuf)
    pltpu.sync_copy(dst2_hbm.at[pl.ds(wid * CPT, CPT)], dbuf)

    iota = lax.iota(jnp.int32, L)
    bufs = ((rows0, gsem0), (rows1, gsem1))

    def _process(rows, ci, p):
        lo = p * HRNG
        for gi in range(K // L):
            didx = dbuf[ci, pl.ds(gi * L, L)]
            rel = didx - lo
            for j in range(L):
                dj = rel.at[jnp.full((L,), j, jnp.int32)].get(
                    mode="promise_in_bounds")
                ok = jnp.logical_and(dj >= 0, dj < HRNG)
                addr = dj * HID + iota
                row = rows[gi * L + j, :]
                plsc.addupdate_scatter(accf, [addr], row, mask=ok)

    def _pass(p):
        @pl.loop(0, ACCH, unroll=8)
        def _zero(i):
            accf[pl.ds(i * L, L)] = jnp.zeros((L,), jnp.float32)

        pltpu.async_copy(g_hbm.at[sbuf.at[0]], rows0, gsem0)

        @pl.loop(0, CPT, step=2)
        def _chunk(ci):
            for par in range(2):
                cc = ci + par
                rows, gsem = bufs[par]
                orows, ogsem = bufs[1 - par]
                pltpu.make_async_copy(g_hbm.at[sbuf.at[cc]], rows, gsem).wait()

                @pl.when(cc + 1 < CPT)
                def _():
                    pltpu.async_copy(g_hbm.at[sbuf.at[cc + 1]], orows, ogsem)

                _process(rows, cc, p)

        pltpu.sync_copy(accf, out_hbm.at[wid, p])

    _pass(0)
    _pass(1)


def _sc_mesh():
    return plsc.VectorSubcoreMesh(core_axis_name="c", subcore_axis_name="s")


def _deg_call(dst2):
    fn = pl.kernel(
        _deg_body,
        out_type=jax.ShapeDtypeStruct((NW, ACC_ROWS), jnp.float32),
        mesh=_sc_mesh(),
        scratch_types=[
            pltpu.VMEM((CPT, K), jnp.int32),
            pltpu.VMEM((ACC_ROWS,), jnp.float32),
        ],
        compiler_params=pltpu.CompilerParams(needs_layout_passes=False),
    )
    return fn(dst2)


def _agg_call(g, src2, dst2):
    fn = pl.kernel(
        _agg_body,
        out_type=jax.ShapeDtypeStruct((NW, 2, HRNG * HID), jnp.float32),
        mesh=_sc_mesh(),
        scratch_types=[
            pltpu.VMEM((CPT, K), jnp.int32),
            pltpu.VMEM((CPT, K), jnp.int32),
            pltpu.VMEM((K, HID), jnp.float32),
            pltpu.VMEM((K, HID), jnp.float32),
            pltpu.VMEM((HRNG * HID,), jnp.float32),
            pltpu.SemaphoreType.DMA,
            pltpu.SemaphoreType.DMA,
        ],
        compiler_params=pltpu.CompilerParams(
            needs_layout_passes=False, use_tc_tiling_on_sc=False),
    )
    return fn(g, src2, dst2)


def _tc1(x_ref, w1_ref, degt_ref, g1_ref):
    deg = jnp.sum(degt_ref[...], axis=1, keepdims=True) + 1.0
    dis = lax.rsqrt(deg)
    h = jnp.dot(x_ref[...], w1_ref[...], preferred_element_type=jnp.float32,
                precision=lax.Precision.HIGHEST)
    g1_ref[0:N_NODES, :] = h * dis[0:N_NODES]
    g1_ref[N_NODES:G_ROWS, :] = jnp.zeros((G_ROWS - N_NODES, HID), jnp.float32)


def _tc2(agg_ref, g1_ref, degt_ref, b1_ref, g2_ref):
    k = pl.program_id(0)
    part = agg_ref[0, 0:N_NODES, :]

    @pl.when(k == 0)
    def _():
        g2_ref[0:N_NODES, :] = g1_ref[0:N_NODES, :] + part
        g2_ref[N_NODES:G_ROWS, :] = jnp.zeros(
            (G_ROWS - N_NODES, HID), jnp.float32)

    @pl.when(k > 0)
    def _():
        g2_ref[0:N_NODES, :] = g2_ref[0:N_NODES, :] + part

    @pl.when(k == NW - 1)
    def _():
        deg = jnp.sum(degt_ref[...], axis=1, keepdims=True) + 1.0
        dis = lax.rsqrt(deg)[0:N_NODES]
        h1 = jnp.maximum(dis * g2_ref[0:N_NODES, :] + b1_ref[...], 0.0)
        g2_ref[0:N_NODES, :] = dis * h1


def _tc3(agg_ref, g2_ref, degt_ref, w2_ref, b2_ref, out_ref, acc_ref):
    k = pl.program_id(0)
    part = agg_ref[0, 0:N_NODES, :]

    @pl.when(k == 0)
    def _():
        acc_ref[...] = g2_ref[0:N_NODES, :] + part

    @pl.when(k > 0)
    def _():
        acc_ref[...] = acc_ref[...] + part

    @pl.when(k == NW - 1)
    def _():
        deg = jnp.sum(degt_ref[...], axis=1, keepdims=True) + 1.0
        dis = lax.rsqrt(deg)[0:N_NODES]
        pre = dis * acc_ref[...]
        out_ref[...] = (
            jnp.dot(pre, w2_ref[...], preferred_element_type=jnp.float32,
                    precision=lax.Precision.HIGHEST)
            + b2_ref[...]
        )


@jax.jit
def kernel(x, edge_index, W1, b1, W2, b2):
    src = edge_index[0]
    dst = edge_index[1]
    pad = E_PAD - N_EDGES
    src2 = jnp.concatenate(
        [src, jnp.full((pad,), PAD_SRC, jnp.int32)]).reshape(NCHUNKS, K)
    dst2 = jnp.concatenate(
        [dst, jnp.full((pad,), PAD_DST, jnp.int32)]).reshape(NCHUNKS, K)

    deg_part = _deg_call(dst2)
    degt = deg_part.T

    g1 = pl.pallas_call(
        _tc1,
        out_shape=jax.ShapeDtypeStruct((G_ROWS, HID), jnp.float32),
    )(x, W1, degt)

    agg1 = _agg_call(g1, src2, dst2).reshape(NW, ACC_ROWS, HID)

    g2 = pl.pallas_call(
        _tc2,
        grid=(NW,),
        in_specs=[
            pl.BlockSpec((1, ACC_ROWS, HID), lambda k: (k, 0, 0)),
            pl.BlockSpec((G_ROWS, HID), lambda k: (0, 0)),
            pl.BlockSpec((ACC_ROWS, NW), lambda k: (0, 0)),
            pl.BlockSpec((1, HID), lambda k: (0, 0)),
        ],
        out_specs=pl.BlockSpec((G_ROWS, HID), lambda k: (0, 0)),
        out_shape=jax.ShapeDtypeStruct((G_ROWS, HID), jnp.float32),
    )(agg1, g1, degt, b1.reshape(1, HID))

    agg2 = _agg_call(g2, src2, dst2).reshape(NW, ACC_ROWS, HID)

    out = pl.pallas_call(
        _tc3,
        grid=(NW,),
        in_specs=[
            pl.BlockSpec((1, ACC_ROWS, HID), lambda k: (k, 0, 0)),
            pl.BlockSpec((G_ROWS, HID), lambda k: (0, 0)),
            pl.BlockSpec((ACC_ROWS, NW), lambda k: (0, 0)),
            pl.BlockSpec((HID, OUT_CH), lambda k: (0, 0)),
            pl.BlockSpec((1, OUT_CH), lambda k: (0, 0)),
        ],
        out_specs=pl.BlockSpec((N_NODES, OUT_CH), lambda k: (0, 0)),
        out_shape=jax.ShapeDtypeStruct((N_NODES, OUT_CH), jnp.float32),
        scratch_shapes=[pltpu.VMEM((N_NODES, HID), jnp.float32)],
    )(agg2, g2, degt, W2, b2.reshape(1, OUT_CH))
    return out


# speedup vs baseline: 1.0216x; 1.0216x over previous
import functools

import jax
import jax.numpy as jnp
from jax import lax
from jax.experimental import pallas as pl
from jax.experimental.pallas import tpu as pltpu
from jax.experimental.pallas import tpu_sc as plsc

N_NODES = 10000
N_EDGES = 320000
IN_CH = 128
HID = 16
OUT_CH = 128

NC = 2
NS = 16
NW = NC * NS
L = 16

K = 128
CPT = 80
E_PAD = NW * CPT * K
NCHUNKS = E_PAD // K
G_ROWS = N_NODES + 8
PAD_SRC = N_NODES
ACC_ROWS = 10112
PAD_DST = N_NODES + 8
RPT = ACC_ROWS // NS
HRNG = ACC_ROWS // 2


def _deg_body(dst2_hbm, deg_hbm, dbuf, hist):
    c = lax.axis_index("c")
    s = lax.axis_index("s")
    wid = c * NS + s

    @pl.loop(0, ACC_ROWS // L)
    def _zero(i):
        hist[pl.ds(i * L, L)] = jnp.zeros((L,), jnp.float32)

    pltpu.sync_copy(dst2_hbm.at[pl.ds(wid * CPT, CPT)], dbuf)

    @pl.loop(0, CPT)
    def _chunk(ci):
        for gi in range(K // L):
            idx = dbuf[ci, pl.ds(gi * L, L)]
            plsc.addupdate_scatter(hist, [idx], jnp.full((L,), 1.0, jnp.float32))

    pltpu.sync_copy(hist, deg_hbm.at[wid])


def _agg_body(g_hbm, src2_hbm, dst2_hbm, out_hbm, sbuf, dbuf, rows0, rows1,
              acc, gsem0, gsem1):
    c = lax.axis_index("c")
    s = lax.axis_index("s")
    wid = c * NS + s

    pltpu.sync_copy(src2_hbm.at[pl.ds(wid * CPT, CPT)], sbuf)
    pltpu.sync_copy(dst2_hbm.at[pl.ds(wid * CPT, CPT)], dbuf)

    iota = lax.iota(jnp.int32, L)
    bufs = ((rows0, gsem0), (rows1, gsem1))

    def _process(rows, ci, p):
        lo = p * HRNG
        for gi in range(K // L):
            didx = dbuf[ci, pl.ds(gi * L, L)]
            rel = didx - lo
            msk = plsc.bitcast(rel, jnp.uint32) < jnp.uint32(HRNG)
            eidx = gi * L + iota
            vals = [
                plsc.load_gather(rows, [eidx, jnp.full((L,), ch, jnp.int32)])
                for ch in range(HID)
            ]
            for ch in range(HID):
                plsc.addupdate_scatter(
                    acc, [rel, jnp.full((L,), ch, jnp.int32)], vals[ch],
                    mask=msk)

    def _pass(p):
        @pl.loop(0, HRNG, unroll=8)
        def _zero(i):
            acc[i, :] = jnp.zeros((L,), jnp.float32)

        pltpu.async_copy(g_hbm.at[sbuf.at[0]], rows0, gsem0)

        @pl.loop(0, CPT, step=2)
        def _chunk(ci):
            for par in range(2):
                cc = ci + par
                rows, gsem = bufs[par]
                orows, ogsem = bufs[1 - par]
                pltpu.make_async_copy(g_hbm.at[sbuf.at[cc]], rows, gsem).wait()

                @pl.when(cc + 1 < CPT)
                def _():
                    pltpu.async_copy(g_hbm.at[sbuf.at[cc + 1]], orows, ogsem)

                _process(rows, cc, p)

        pltpu.sync_copy(acc, out_hbm.at[wid, p])

    _pass(0)
    _pass(1)


def _sc_mesh():
    return plsc.VectorSubcoreMesh(core_axis_name="c", subcore_axis_name="s")


def _deg_call(dst2):
    fn = pl.kernel(
        _deg_body,
        out_type=jax.ShapeDtypeStruct((NW, ACC_ROWS), jnp.float32),
        mesh=_sc_mesh(),
        scratch_types=[
            pltpu.VMEM((CPT, K), jnp.int32),
            pltpu.VMEM((ACC_ROWS,), jnp.float32),
        ],
        compiler_params=pltpu.CompilerParams(needs_layout_passes=False),
    )
    return fn(dst2)


def _agg_call(g, src2, dst2):
    fn = pl.kernel(
        _agg_body,
        out_type=jax.ShapeDtypeStruct((NW, 2, HRNG, HID), jnp.float32),
        mesh=_sc_mesh(),
        scratch_types=[
            pltpu.VMEM((CPT, K), jnp.int32),
            pltpu.VMEM((CPT, K), jnp.int32),
            pltpu.VMEM((K, HID), jnp.float32),
            pltpu.VMEM((K, HID), jnp.float32),
            pltpu.VMEM((HRNG, HID), jnp.float32),
            pltpu.SemaphoreType.DMA,
            pltpu.SemaphoreType.DMA,
        ],
        compiler_params=pltpu.CompilerParams(
            needs_layout_passes=False, use_tc_tiling_on_sc=False),
    )
    return fn(g, src2, dst2)


def _tc1(x_ref, w1_ref, degt_ref, g1_ref):
    deg = jnp.sum(degt_ref[...], axis=1, keepdims=True) + 1.0
    dis = lax.rsqrt(deg)
    h = jnp.dot(x_ref[...], w1_ref[...], preferred_element_type=jnp.float32,
                precision=lax.Precision.HIGHEST)
    g1_ref[0:N_NODES, :] = h * dis[0:N_NODES]
    g1_ref[N_NODES:G_ROWS, :] = jnp.zeros((G_ROWS - N_NODES, HID), jnp.float32)


def _tc2(agg_ref, g1_ref, degt_ref, b1_ref, g2_ref):
    k = pl.program_id(0)
    part = agg_ref[0, 0:N_NODES, :]

    @pl.when(k == 0)
    def _():
        g2_ref[0:N_NODES, :] = g1_ref[0:N_NODES, :] + part
        g2_ref[N_NODES:G_ROWS, :] = jnp.zeros(
            (G_ROWS - N_NODES, HID), jnp.float32)

    @pl.when(k > 0)
    def _():
        g2_ref[0:N_NODES, :] = g2_ref[0:N_NODES, :] + part

    @pl.when(k == NW - 1)
    def _():
        deg = jnp.sum(degt_ref[...], axis=1, keepdims=True) + 1.0
        dis = lax.rsqrt(deg)[0:N_NODES]
        h1 = jnp.maximum(dis * g2_ref[0:N_NODES, :] + b1_ref[...], 0.0)
        g2_ref[0:N_NODES, :] = dis * h1


def _tc3(agg_ref, g2_ref, degt_ref, w2_ref, b2_ref, out_ref, acc_ref):
    k = pl.program_id(0)
    part = agg_ref[0, 0:N_NODES, :]

    @pl.when(k == 0)
    def _():
        acc_ref[...] = g2_ref[0:N_NODES, :] + part

    @pl.when(k > 0)
    def _():
        acc_ref[...] = acc_ref[...] + part

    @pl.when(k == NW - 1)
    def _():
        deg = jnp.sum(degt_ref[...], axis=1, keepdims=True) + 1.0
        dis = lax.rsqrt(deg)[0:N_NODES]
        pre = dis * acc_ref[...]
        out_ref[...] = (
            jnp.dot(pre, w2_ref[...], preferred_element_type=jnp.float32,
                    precision=lax.Precision.HIGHEST)
            + b2_ref[...]
        )


@jax.jit
def kernel(x, edge_index, W1, b1, W2, b2):
    src = edge_index[0]
    dst = edge_index[1]
    pad = E_PAD - N_EDGES
    src2 = jnp.concatenate(
        [src, jnp.full((pad,), PAD_SRC, jnp.int32)]).reshape(NCHUNKS, K)
    dst2 = jnp.concatenate(
        [dst, jnp.full((pad,), PAD_DST, jnp.int32)]).reshape(NCHUNKS, K)

    deg_part = _deg_call(dst2)
    degt = deg_part.T

    g1 = pl.pallas_call(
        _tc1,
        out_shape=jax.ShapeDtypeStruct((G_ROWS, HID), jnp.float32),
    )(x, W1, degt)

    agg1 = _agg_call(g1, src2, dst2).reshape(NW, ACC_ROWS, HID)

    g2 = pl.pallas_call(
        _tc2,
        grid=(NW,),
        in_specs=[
            pl.BlockSpec((1, ACC_ROWS, HID), lambda k: (k, 0, 0)),
            pl.BlockSpec((G_ROWS, HID), lambda k: (0, 0)),
            pl.BlockSpec((ACC_ROWS, NW), lambda k: (0, 0)),
            pl.BlockSpec((1, HID), lambda k: (0, 0)),
        ],
        out_specs=pl.BlockSpec((G_ROWS, HID), lambda k: (0, 0)),
        out_shape=jax.ShapeDtypeStruct((G_ROWS, HID), jnp.float32),
    )(agg1, g1, degt, b1.reshape(1, HID))

    agg2 = _agg_call(g2, src2, dst2).reshape(NW, ACC_ROWS, HID)

    out = pl.pallas_call(
        _tc3,
        grid=(NW,),
        in_specs=[
            pl.BlockSpec((1, ACC_ROWS, HID), lambda k: (k, 0, 0)),
            pl.BlockSpec((G_ROWS, HID), lambda k: (0, 0)),
            pl.BlockSpec((ACC_ROWS, NW), lambda k: (0, 0)),
            pl.BlockSpec((HID, OUT_CH), lambda k: (0, 0)),
            pl.BlockSpec((1, OUT_CH), lambda k: (0, 0)),
        ],
        out_specs=pl.BlockSpec((N_NODES, OUT_CH), lambda k: (0, 0)),
        out_shape=jax.ShapeDtypeStruct((N_NODES, OUT_CH), jnp.float32),
        scratch_shapes=[pltpu.VMEM((N_NODES, HID), jnp.float32)],
    )(agg2, g2, degt, W2, b2.reshape(1, OUT_CH))
    return out


# speedup vs baseline: 3.2157x; 3.1476x over previous
import functools

import jax
import jax.numpy as jnp
from jax import lax
from jax.experimental import pallas as pl
from jax.experimental.pallas import tpu as pltpu
from jax.experimental.pallas import tpu_sc as plsc

N_NODES = 10000
N_EDGES = 320000
IN_CH = 128
HID = 16
OUT_CH = 128

NC = 2
NS = 16
NW = NC * NS
L = 16

K = 128
CPT = 80
E_PAD = NW * CPT * K
NCHUNKS = E_PAD // K
G_ROWS = N_NODES + 8
PAD_SRC = N_NODES
ACC_ROWS = 10112
PAD_DST = N_NODES + 8
RPT = ACC_ROWS // NS


def _deg_body(dst2_hbm, deg_hbm, dbuf, hist):
    c = lax.axis_index("c")
    s = lax.axis_index("s")
    wid = c * NS + s

    @pl.loop(0, ACC_ROWS // L)
    def _zero(i):
        hist[pl.ds(i * L, L)] = jnp.zeros((L,), jnp.float32)

    pltpu.sync_copy(dst2_hbm.at[pl.ds(wid * CPT, CPT)], dbuf)

    @pl.loop(0, CPT)
    def _chunk(ci):
        for gi in range(K // L):
            idx = dbuf[ci, pl.ds(gi * L, L)]
            plsc.addupdate_scatter(hist, [idx], jnp.full((L,), 1.0, jnp.float32))

    pltpu.sync_copy(hist, deg_hbm.at[wid])


def _agg_body(g_hbm, src2_hbm, dst2_hbm, out_hbm, sbuf, dbuf, rows0, rows1,
              zbuf, acc, gsem0, gsem1, ssem0, ssem1):
    c = lax.axis_index("c")
    s = lax.axis_index("s")
    wid = c * NS + s

    @pl.loop(0, RPT)
    def _zero(i):
        zbuf[i, :] = jnp.zeros((L,), jnp.float32)

    pltpu.sync_copy(zbuf, acc.at[pl.ds(s * RPT, RPT)])
    pltpu.sync_copy(src2_hbm.at[pl.ds(wid * CPT, CPT)], sbuf)
    pltpu.sync_copy(dst2_hbm.at[pl.ds(wid * CPT, CPT)], dbuf)
    plsc.subcore_barrier()

    bufs = ((rows0, gsem0, ssem0), (rows1, gsem1, ssem1))

    pltpu.async_copy(g_hbm.at[sbuf.at[0]], rows0, gsem0)

    def _step(ci, par):
        rows, gsem, ssem = bufs[par]
        orows, ogsem, ossem = bufs[1 - par]
        pltpu.make_async_copy(g_hbm.at[sbuf.at[ci]], rows, gsem).wait()

        @pl.when(ci >= 1)
        def _():
            pltpu.make_async_copy(orows, acc.at[dbuf.at[ci - 1]], ossem).wait()

        @pl.when(ci + 1 < CPT)
        def _():
            pltpu.async_copy(g_hbm.at[sbuf.at[ci + 1]], orows, ogsem)

        pltpu.async_copy(rows, acc.at[dbuf.at[ci]], ssem, add=True)

    @pl.loop(0, CPT, step=2)
    def _chunk(ci):
        _step(ci, 0)
        _step(ci + 1, 1)

    pltpu.make_async_copy(rows1, acc.at[dbuf.at[CPT - 1]], ssem1).wait()

    plsc.subcore_barrier()
    pltpu.sync_copy(acc.at[pl.ds(s * RPT, RPT)], out_hbm.at[c, pl.ds(s * RPT, RPT)])


def _sc_mesh():
    return plsc.VectorSubcoreMesh(core_axis_name="c", subcore_axis_name="s")


def _deg_call(dst2):
    fn = pl.kernel(
        _deg_body,
        out_type=jax.ShapeDtypeStruct((NW, ACC_ROWS), jnp.float32),
        mesh=_sc_mesh(),
        scratch_types=[
            pltpu.VMEM((CPT, K), jnp.int32),
            pltpu.VMEM((ACC_ROWS,), jnp.float32),
        ],
        compiler_params=pltpu.CompilerParams(needs_layout_passes=False),
    )
    return fn(dst2)


def _agg_call(g, src2, dst2):
    fn = pl.kernel(
        _agg_body,
        out_type=jax.ShapeDtypeStruct((NC, ACC_ROWS, HID), jnp.float32),
        mesh=_sc_mesh(),
        scratch_types=[
            pltpu.VMEM((CPT, K), jnp.int32),
            pltpu.VMEM((CPT, K), jnp.int32),
            pltpu.VMEM((K, HID), jnp.float32),
            pltpu.VMEM((K, HID), jnp.float32),
            pltpu.VMEM((RPT, HID), jnp.float32),
            pltpu.VMEM_SHARED((ACC_ROWS, HID), jnp.float32),
            pltpu.SemaphoreType.DMA,
            pltpu.SemaphoreType.DMA,
            pltpu.SemaphoreType.DMA,
            pltpu.SemaphoreType.DMA,
        ],
        compiler_params=pltpu.CompilerParams(use_tc_tiling_on_sc=False),
    )
    return fn(g, src2, dst2)


def _tc1(x_ref, w1_ref, degt_ref, g1_ref):
    deg = jnp.sum(degt_ref[...], axis=1, keepdims=True) + 1.0
    dis = lax.rsqrt(deg)
    h = jnp.dot(x_ref[...], w1_ref[...], preferred_element_type=jnp.float32,
                precision=lax.Precision.HIGHEST)
    g1_ref[0:N_NODES, :] = h * dis[0:N_NODES]
    g1_ref[N_NODES:G_ROWS, :] = jnp.zeros((G_ROWS - N_NODES, HID), jnp.float32)


def _tc2(agg_ref, g1_ref, degt_ref, b1_ref, g2_ref):
    deg = jnp.sum(degt_ref[...], axis=1, keepdims=True) + 1.0
    dis = lax.rsqrt(deg)[0:N_NODES]
    ssum = agg_ref[0, 0:N_NODES, :] + agg_ref[1, 0:N_NODES, :] + g1_ref[0:N_NODES, :]
    h1 = jnp.maximum(dis * ssum + b1_ref[...], 0.0)
    g2_ref[0:N_NODES, :] = dis * h1
    g2_ref[N_NODES:G_ROWS, :] = jnp.zeros((G_ROWS - N_NODES, HID), jnp.float32)


def _tc3(agg_ref, g2_ref, degt_ref, w2_ref, b2_ref, out_ref):
    deg = jnp.sum(degt_ref[...], axis=1, keepdims=True) + 1.0
    dis = lax.rsqrt(deg)[0:N_NODES]
    ssum = agg_ref[0, 0:N_NODES, :] + agg_ref[1, 0:N_NODES, :] + g2_ref[0:N_NODES, :]
    pre = dis * ssum
    out_ref[...] = (
        jnp.dot(pre, w2_ref[...], preferred_element_type=jnp.float32,
                precision=lax.Precision.HIGHEST)
        + b2_ref[...]
    )


@jax.jit
def kernel(x, edge_index, W1, b1, W2, b2):
    src = edge_index[0]
    dst = edge_index[1]
    pad = E_PAD - N_EDGES
    src2 = jnp.concatenate(
        [src, jnp.full((pad,), PAD_SRC, jnp.int32)]).reshape(NCHUNKS, K)
    dst2 = jnp.concatenate(
        [dst, jnp.full((pad,), PAD_DST, jnp.int32)]).reshape(NCHUNKS, K)

    deg_part = _deg_call(dst2)
    degt = deg_part.T

    g1 = pl.pallas_call(
        _tc1,
        out_shape=jax.ShapeDtypeStruct((G_ROWS, HID), jnp.float32),
    )(x, W1, degt)

    agg1 = _agg_call(g1, src2, dst2)

    g2 = pl.pallas_call(
        _tc2,
        out_shape=jax.ShapeDtypeStruct((G_ROWS, HID), jnp.float32),
    )(agg1, g1, degt, b1.reshape(1, HID))

    agg2 = _agg_call(g2, src2, dst2)

    out = pl.pallas_call(
        _tc3,
        out_shape=jax.ShapeDtypeStruct((N_NODES, OUT_CH), jnp.float32),
    )(agg2, g2, degt, W2, b2.reshape(1, OUT_CH))
    return out
